# Initial kernel scaffold; baseline (speedup 1.0000x reference)
import jax
import jax.numpy as jnp
from jax.experimental import pallas as pl


def kernel(features, edge_index, W1, b1, W2, b2, W3, b3):
    raise NotImplementedError("write your pallas kernel here")


# speedup vs baseline: 3.9912x; 3.9912x over previous
import functools

import jax
import jax.numpy as jnp
from jax import lax
from jax.experimental import pallas as pl
from jax.experimental.pallas import tpu as pltpu
from jax.experimental.pallas import tpu_sc as plsc

N_NODES = 10000
N_PAD = 10240
N_EDGES = 320000
F = 128
FH = 64
N_CLASSES = 40

NC = 2
NS = 16
CHUNK = 128
E_PAD = 327680
NCHUNK = E_PAD // NS // CHUNK
RPS = N_PAD // NS
DEG_W = 16

_mesh = plsc.VectorSubcoreMesh(core_axis_name="c", subcore_axis_name="s")
_sc_params = pltpu.CompilerParams(use_tc_tiling_on_sc=False)


def _agg_body(y_hbm, src_hbm, dst_hbm, zeros_hbm, out_hbm,
              srcv, dstv, rows0, rows1, acc, sem0, sem1):
    c = lax.axis_index("c")
    s = lax.axis_index("s")
    pltpu.sync_copy(src_hbm.at[s], srcv)
    pltpu.sync_copy(dst_hbm.at[s], dstv)
    sl = pl.ds(s * RPS, RPS)
    pltpu.sync_copy(zeros_hbm.at[sl], acc.at[sl])
    plsc.subcore_barrier()

    yc = y_hbm.at[c]
    pltpu.async_copy(yc.at[srcv.at[0]], rows0, sem0)

    @pl.loop(0, NCHUNK, step=2)
    def _(j):
        pltpu.make_async_copy(yc.at[srcv.at[0]], rows0, sem0).wait()
        pltpu.async_copy(yc.at[srcv.at[j + 1]], rows1, sem1)
        pltpu.sync_copy(rows0, acc.at[dstv.at[j]], add=True)
        pltpu.make_async_copy(yc.at[srcv.at[0]], rows1, sem1).wait()

        @pl.when(j + 2 < NCHUNK)
        def _():
            pltpu.async_copy(yc.at[srcv.at[j + 2]], rows0, sem0)
        pltpu.sync_copy(rows1, acc.at[dstv.at[j + 1]], add=True)

    plsc.subcore_barrier()
    pltpu.sync_copy(acc.at[sl], out_hbm.at[c].at[sl])


@functools.partial(
    pl.kernel, mesh=_mesh,
    out_type=jax.ShapeDtypeStruct((NC, N_PAD, FH), jnp.float32),
    scratch_types=[
        pltpu.VMEM((NCHUNK, CHUNK), jnp.int32),
        pltpu.VMEM((NCHUNK, CHUNK), jnp.int32),
        pltpu.VMEM((CHUNK, FH), jnp.float32),
        pltpu.VMEM((CHUNK, FH), jnp.float32),
        pltpu.VMEM_SHARED((N_PAD, FH), jnp.float32),
        pltpu.SemaphoreType.DMA,
        pltpu.SemaphoreType.DMA,
    ],
    compiler_params=_sc_params,
)
def _sc_aggregate(*args):
    _agg_body(*args)


def _deg_body(src_hbm, dst_hbm, ones_hbm, zeros_hbm, od_hbm, id_hbm,
              idxv, ones_v, acc):
    c = lax.axis_index("c")
    s = lax.axis_index("s")

    @pl.when(c == 0)
    def _():
        pltpu.sync_copy(src_hbm.at[s], idxv)

    @pl.when(c == 1)
    def _():
        pltpu.sync_copy(dst_hbm.at[s], idxv)
    pltpu.sync_copy(ones_hbm, ones_v)
    sl = pl.ds(s * RPS, RPS)
    pltpu.sync_copy(zeros_hbm.at[sl], acc.at[sl])
    plsc.subcore_barrier()

    @pl.loop(0, NCHUNK)
    def _(j):
        pltpu.sync_copy(ones_v, acc.at[idxv.at[j]], add=True)

    plsc.subcore_barrier()

    @pl.when(c == 0)
    def _():
        pltpu.sync_copy(acc.at[sl], od_hbm.at[sl])

    @pl.when(c == 1)
    def _():
        pltpu.sync_copy(acc.at[sl], id_hbm.at[sl])


@functools.partial(
    pl.kernel, mesh=_mesh,
    out_type=(jax.ShapeDtypeStruct((N_PAD, DEG_W), jnp.float32),
              jax.ShapeDtypeStruct((N_PAD, DEG_W), jnp.float32)),
    scratch_types=[
        pltpu.VMEM((NCHUNK, CHUNK), jnp.int32),
        pltpu.VMEM((CHUNK, DEG_W), jnp.float32),
        pltpu.VMEM_SHARED((N_PAD, DEG_W), jnp.float32),
    ],
    compiler_params=_sc_params,
)
def _sc_degrees(*args):
    _deg_body(*args)


def _halves(o_ref, t):
    o_ref[0] = t[:, :FH]
    o_ref[1] = t[:, FH:]


def _rs(ref):
    return lax.rsqrt(jnp.maximum(ref[:, 0:1], 1.0))


def _l1_body(x_ref, w_ref, od_ref, o_ref):
    xw = jnp.dot(x_ref[...], w_ref[...], preferred_element_type=jnp.float32)
    _halves(o_ref, xw * _rs(od_ref))


def _mid_body(g_ref, od_ref, id_ref, b_ref, w_ref, o_ref):
    si = _rs(id_ref)
    so = _rs(od_ref)
    h0 = jnp.maximum(g_ref[0] * si + b_ref[:, :FH], 0.0) * so
    h1 = jnp.maximum(g_ref[1] * si + b_ref[:, FH:], 0.0) * so
    t = (jnp.dot(h0, w_ref[:FH, :], preferred_element_type=jnp.float32)
         + jnp.dot(h1, w_ref[FH:, :], preferred_element_type=jnp.float32))
    _halves(o_ref, t)


def _pre3_body(g_ref, od_ref, id_ref, b_ref, o_ref):
    si = _rs(id_ref)
    so = _rs(od_ref)
    o_ref[0] = jnp.maximum(g_ref[0] * si + b_ref[:, :FH], 0.0) * so
    o_ref[1] = jnp.maximum(g_ref[1] * si + b_ref[:, FH:], 0.0) * so


def _final_body(g_ref, id_ref, w_ref, b_ref, o_ref):
    si = _rs(id_ref)
    o_ref[...] = (
        jnp.dot(g_ref[0] * si, w_ref[:FH, :], preferred_element_type=jnp.float32)
        + jnp.dot(g_ref[1] * si, w_ref[FH:, :], preferred_element_type=jnp.float32)
        + b_ref[...])


def _tc(body, out_shape, *args):
    return pl.pallas_call(
        body, out_shape=jax.ShapeDtypeStruct(out_shape, jnp.float32))(*args)


def kernel(features, edge_index, W1, b1, W2, b2, W3, b3):
    src = edge_index[0].astype(jnp.int32)
    dst = edge_index[1].astype(jnp.int32)
    pad = jnp.full((E_PAD - N_EDGES,), N_PAD - 1, jnp.int32)
    srcp = jnp.concatenate([src, pad]).reshape(NS, NCHUNK, CHUNK)
    dstp = jnp.concatenate([dst, pad]).reshape(NS, NCHUNK, CHUNK)
    x = jnp.pad(features, ((0, N_PAD - N_NODES), (0, 0)))

    zeros_f = jnp.zeros((N_PAD, FH), jnp.float32)
    zeros_d = jnp.zeros((N_PAD, DEG_W), jnp.float32)
    ones_d = jnp.ones((CHUNK, DEG_W), jnp.float32)
    b1r = b1[None, :]
    b2r = b2[None, :]
    W3p = jnp.pad(W3, ((0, 0), (0, F - N_CLASSES)))
    b3r = jnp.pad(b3, (0, F - N_CLASSES))[None, :]

    od, idg = _sc_degrees(srcp, dstp, ones_d, zeros_d)

    y0 = _tc(_l1_body, (NC, N_PAD, FH), x, W1, od)
    g1 = _sc_aggregate(y0, srcp, dstp, zeros_f)
    y1 = _tc(_mid_body, (NC, N_PAD, FH), g1, od, idg, b1r, W2)
    g2 = _sc_aggregate(y1, srcp, dstp, zeros_f)
    y2 = _tc(_pre3_body, (NC, N_PAD, FH), g2, od, idg, b2r)
    g3 = _sc_aggregate(y2, srcp, dstp, zeros_f)
    out = _tc(_final_body, (N_PAD, F), g3, idg, W3p, b3r)
    return out[:N_NODES, :N_CLASSES]


# speedup vs baseline: 4.3644x; 1.0935x over previous
import functools

import jax
import jax.numpy as jnp
from jax import lax
from jax.experimental import pallas as pl
from jax.experimental.pallas import tpu as pltpu
from jax.experimental.pallas import tpu_sc as plsc

N_NODES = 10000
N_PAD = 10240
N_EDGES = 320000
F = 128
FH = 64
N_CLASSES = 40

NC = 2
NS = 16
CHUNK = 256
E_PAD = 327680
NCHUNK = E_PAD // NS // CHUNK
RPS = N_PAD // NS
DEG_W = 16

_mesh = plsc.VectorSubcoreMesh(core_axis_name="c", subcore_axis_name="s")
_sc_params = pltpu.CompilerParams(use_tc_tiling_on_sc=False)


def _agg_body(y_hbm, src_hbm, dst_hbm, zeros_hbm, out_hbm,
              srcv, dstv, rows0, rows1, acc, sem0, sem1):
    c = lax.axis_index("c")
    s = lax.axis_index("s")
    pltpu.sync_copy(src_hbm.at[s], srcv)
    pltpu.sync_copy(dst_hbm.at[s], dstv)
    sl = pl.ds(s * RPS, RPS)
    pltpu.sync_copy(zeros_hbm.at[sl], acc.at[sl])
    plsc.subcore_barrier()

    yc = y_hbm.at[c]
    pltpu.async_copy(yc.at[srcv.at[0]], rows0, sem0)

    @pl.loop(0, NCHUNK, step=2)
    def _(j):
        pltpu.make_async_copy(yc.at[srcv.at[0]], rows0, sem0).wait()
        pltpu.async_copy(yc.at[srcv.at[j + 1]], rows1, sem1)
        pltpu.sync_copy(rows0, acc.at[dstv.at[j]], add=True)
        pltpu.make_async_copy(yc.at[srcv.at[0]], rows1, sem1).wait()

        @pl.when(j + 2 < NCHUNK)
        def _():
            pltpu.async_copy(yc.at[srcv.at[j + 2]], rows0, sem0)
        pltpu.sync_copy(rows1, acc.at[dstv.at[j + 1]], add=True)

    plsc.subcore_barrier()
    pltpu.sync_copy(acc.at[sl], out_hbm.at[c].at[sl])


@functools.partial(
    pl.kernel, mesh=_mesh,
    out_type=jax.ShapeDtypeStruct((NC, N_PAD, FH), jnp.float32),
    scratch_types=[
        pltpu.VMEM((NCHUNK, CHUNK), jnp.int32),
        pltpu.VMEM((NCHUNK, CHUNK), jnp.int32),
        pltpu.VMEM((CHUNK, FH), jnp.float32),
        pltpu.VMEM((CHUNK, FH), jnp.float32),
        pltpu.VMEM_SHARED((N_PAD, FH), jnp.float32),
        pltpu.SemaphoreType.DMA,
        pltpu.SemaphoreType.DMA,
    ],
    compiler_params=_sc_params,
)
def _sc_aggregate(*args):
    _agg_body(*args)


def _deg_body(src_hbm, dst_hbm, ones_hbm, zeros_hbm, od_hbm, id_hbm,
              idxv, ones_v, acc):
    c = lax.axis_index("c")
    s = lax.axis_index("s")

    @pl.when(c == 0)
    def _():
        pltpu.sync_copy(src_hbm.at[s], idxv)

    @pl.when(c == 1)
    def _():
        pltpu.sync_copy(dst_hbm.at[s], idxv)
    pltpu.sync_copy(ones_hbm, ones_v)
    sl = pl.ds(s * RPS, RPS)
    pltpu.sync_copy(zeros_hbm.at[sl], acc.at[sl])
    plsc.subcore_barrier()

    @pl.loop(0, NCHUNK)
    def _(j):
        pltpu.sync_copy(ones_v, acc.at[idxv.at[j]], add=True)

    plsc.subcore_barrier()

    @pl.when(c == 0)
    def _():
        pltpu.sync_copy(acc.at[sl], od_hbm.at[sl])

    @pl.when(c == 1)
    def _():
        pltpu.sync_copy(acc.at[sl], id_hbm.at[sl])


@functools.partial(
    pl.kernel, mesh=_mesh,
    out_type=(jax.ShapeDtypeStruct((N_PAD, DEG_W), jnp.float32),
              jax.ShapeDtypeStruct((N_PAD, DEG_W), jnp.float32)),
    scratch_types=[
        pltpu.VMEM((NCHUNK, CHUNK), jnp.int32),
        pltpu.VMEM((CHUNK, DEG_W), jnp.float32),
        pltpu.VMEM_SHARED((N_PAD, DEG_W), jnp.float32),
    ],
    compiler_params=_sc_params,
)
def _sc_degrees(*args):
    _deg_body(*args)


def _halves(o_ref, t):
    o_ref[0] = t[:, :FH]
    o_ref[1] = t[:, FH:]


def _rs(ref):
    return lax.rsqrt(jnp.maximum(ref[:, 0:1], 1.0))


def _l1_body(x_ref, w_ref, od_ref, o_ref):
    xw = jnp.dot(x_ref[...], w_ref[...], preferred_element_type=jnp.float32)
    _halves(o_ref, xw * _rs(od_ref))


def _mid_body(g_ref, od_ref, id_ref, b_ref, w_ref, o_ref):
    si = _rs(id_ref)
    so = _rs(od_ref)
    h0 = jnp.maximum(g_ref[0] * si + b_ref[:, :FH], 0.0) * so
    h1 = jnp.maximum(g_ref[1] * si + b_ref[:, FH:], 0.0) * so
    t = (jnp.dot(h0, w_ref[:FH, :], preferred_element_type=jnp.float32)
         + jnp.dot(h1, w_ref[FH:, :], preferred_element_type=jnp.float32))
    _halves(o_ref, t)


def _pre3_body(g_ref, od_ref, id_ref, b_ref, o_ref):
    si = _rs(id_ref)
    so = _rs(od_ref)
    o_ref[0] = jnp.maximum(g_ref[0] * si + b_ref[:, :FH], 0.0) * so
    o_ref[1] = jnp.maximum(g_ref[1] * si + b_ref[:, FH:], 0.0) * so


def _final_body(g_ref, id_ref, w_ref, b_ref, o_ref):
    si = _rs(id_ref)
    o_ref[...] = (
        jnp.dot(g_ref[0] * si, w_ref[:FH, :], preferred_element_type=jnp.float32)
        + jnp.dot(g_ref[1] * si, w_ref[FH:, :], preferred_element_type=jnp.float32)
        + b_ref[...])


def _tc(body, out_shape, *args):
    return pl.pallas_call(
        body, out_shape=jax.ShapeDtypeStruct(out_shape, jnp.float32))(*args)


def kernel(features, edge_index, W1, b1, W2, b2, W3, b3):
    src = edge_index[0].astype(jnp.int32)
    dst = edge_index[1].astype(jnp.int32)
    pad = jnp.full((E_PAD - N_EDGES,), N_PAD - 1, jnp.int32)
    srcp = jnp.concatenate([src, pad]).reshape(NS, NCHUNK, CHUNK)
    dstp = jnp.concatenate([dst, pad]).reshape(NS, NCHUNK, CHUNK)
    x = jnp.pad(features, ((0, N_PAD - N_NODES), (0, 0)))

    zeros_f = jnp.zeros((N_PAD, FH), jnp.float32)
    zeros_d = jnp.zeros((N_PAD, DEG_W), jnp.float32)
    ones_d = jnp.ones((CHUNK, DEG_W), jnp.float32)
    b1r = b1[None, :]
    b2r = b2[None, :]
    W3p = jnp.pad(W3, ((0, 0), (0, F - N_CLASSES)))
    b3r = jnp.pad(b3, (0, F - N_CLASSES))[None, :]

    od, idg = _sc_degrees(srcp, dstp, ones_d, zeros_d)

    y0 = _tc(_l1_body, (NC, N_PAD, FH), x, W1, od)
    g1 = _sc_aggregate(y0, srcp, dstp, zeros_f)
    y1 = _tc(_mid_body, (NC, N_PAD, FH), g1, od, idg, b1r, W2)
    g2 = _sc_aggregate(y1, srcp, dstp, zeros_f)
    y2 = _tc(_pre3_body, (NC, N_PAD, FH), g2, od, idg, b2r)
    g3 = _sc_aggregate(y2, srcp, dstp, zeros_f)
    out = _tc(_final_body, (N_PAD, F), g3, idg, W3p, b3r)
    return out[:N_NODES, :N_CLASSES]


# speedup vs baseline: 9.3051x; 2.1320x over previous
import functools

import jax
import jax.numpy as jnp
from jax import lax
from jax.experimental import pallas as pl
from jax.experimental.pallas import tpu as pltpu
from jax.experimental.pallas import tpu_sc as plsc

N_NODES = 10000
N_PAD = 10240
N_EDGES = 320000
F = 128
N_CLASSES = 40

NC = 2
NS = 16
CHUNK = 100
NCHUNK = 100
RPS = N_PAD // NS
DEG_W = 16
CD = 200
ND = 100

_mesh = plsc.VectorSubcoreMesh(core_axis_name="c", subcore_axis_name="s")
_sc_params = pltpu.CompilerParams(use_tc_tiling_on_sc=False)


def _agg_body(y_hbm, src_hbm, dst_hbm, zeros_hbm, out_hbm,
              srcv, dstv, rows0, rows1, acc, sem0, sem1):
    c = lax.axis_index("c")
    s = lax.axis_index("s")
    pltpu.sync_copy(src_hbm.at[c].at[s], srcv)
    pltpu.sync_copy(dst_hbm.at[c].at[s], dstv)
    sl = pl.ds(s * RPS, RPS)
    pltpu.sync_copy(zeros_hbm.at[sl], acc.at[sl])
    plsc.subcore_barrier()

    pltpu.async_copy(y_hbm.at[srcv.at[0]], rows0, sem0)

    @pl.loop(0, NCHUNK, step=2)
    def _(j):
        pltpu.make_async_copy(y_hbm.at[srcv.at[0]], rows0, sem0).wait()
        pltpu.async_copy(y_hbm.at[srcv.at[j + 1]], rows1, sem1)
        pltpu.sync_copy(rows0, acc.at[dstv.at[j]], add=True)
        pltpu.make_async_copy(y_hbm.at[srcv.at[0]], rows1, sem1).wait()

        @pl.when(j + 2 < NCHUNK)
        def _():
            pltpu.async_copy(y_hbm.at[srcv.at[j + 2]], rows0, sem0)
        pltpu.sync_copy(rows1, acc.at[dstv.at[j + 1]], add=True)

    plsc.subcore_barrier()
    pltpu.sync_copy(acc.at[sl], out_hbm.at[c].at[sl])


@functools.partial(
    pl.kernel, mesh=_mesh,
    out_type=jax.ShapeDtypeStruct((NC, N_PAD, F), jnp.float32),
    scratch_types=[
        pltpu.VMEM((NCHUNK, CHUNK), jnp.int32),
        pltpu.VMEM((NCHUNK, CHUNK), jnp.int32),
        pltpu.VMEM((CHUNK, F), jnp.float32),
        pltpu.VMEM((CHUNK, F), jnp.float32),
        pltpu.VMEM_SHARED((N_PAD, F), jnp.float32),
        pltpu.SemaphoreType.DMA,
        pltpu.SemaphoreType.DMA,
    ],
    compiler_params=_sc_params,
)
def _sc_aggregate(*args):
    _agg_body(*args)


def _deg_body(src_hbm, dst_hbm, ones_hbm, zeros_hbm, od_hbm, id_hbm,
              idxv, ones_v, acc):
    c = lax.axis_index("c")
    s = lax.axis_index("s")

    @pl.when(c == 0)
    def _():
        pltpu.sync_copy(src_hbm.at[s], idxv)

    @pl.when(c == 1)
    def _():
        pltpu.sync_copy(dst_hbm.at[s], idxv)
    pltpu.sync_copy(ones_hbm, ones_v)
    sl = pl.ds(s * RPS, RPS)
    pltpu.sync_copy(zeros_hbm.at[sl], acc.at[sl])
    plsc.subcore_barrier()

    @pl.loop(0, ND)
    def _(j):
        pltpu.sync_copy(ones_v, acc.at[idxv.at[j]], add=True)

    plsc.subcore_barrier()

    @pl.when(c == 0)
    def _():
        pltpu.sync_copy(acc.at[sl], od_hbm.at[sl])

    @pl.when(c == 1)
    def _():
        pltpu.sync_copy(acc.at[sl], id_hbm.at[sl])


@functools.partial(
    pl.kernel, mesh=_mesh,
    out_type=(jax.ShapeDtypeStruct((N_PAD, DEG_W), jnp.float32),
              jax.ShapeDtypeStruct((N_PAD, DEG_W), jnp.float32)),
    scratch_types=[
        pltpu.VMEM((ND, CD), jnp.int32),
        pltpu.VMEM((CD, DEG_W), jnp.float32),
        pltpu.VMEM_SHARED((N_PAD, DEG_W), jnp.float32),
    ],
    compiler_params=_sc_params,
)
def _sc_degrees(*args):
    _deg_body(*args)


def _rs(ref):
    return lax.rsqrt(jnp.maximum(ref[:, 0:1], 1.0))


def _l1_body(x_ref, w_ref, od_ref, o_ref):
    xw = jnp.dot(x_ref[...], w_ref[...], preferred_element_type=jnp.float32)
    o_ref[...] = xw * _rs(od_ref)


def _mid_body(g_ref, od_ref, id_ref, b_ref, w_ref, o_ref):
    g = g_ref[0] + g_ref[1]
    h = jnp.maximum(g * _rs(id_ref) + b_ref[...], 0.0) * _rs(od_ref)
    o_ref[...] = jnp.dot(h, w_ref[...], preferred_element_type=jnp.float32)


def _pre3_body(g_ref, od_ref, id_ref, b_ref, o_ref):
    g = g_ref[0] + g_ref[1]
    o_ref[...] = jnp.maximum(g * _rs(id_ref) + b_ref[...], 0.0) * _rs(od_ref)


def _final_body(g_ref, id_ref, w_ref, b_ref, o_ref):
    g = (g_ref[0] + g_ref[1]) * _rs(id_ref)
    o_ref[...] = (jnp.dot(g, w_ref[...], preferred_element_type=jnp.float32)
                  + b_ref[...])


def _tc(body, out_shape, *args):
    return pl.pallas_call(
        body, out_shape=jax.ShapeDtypeStruct(out_shape, jnp.float32))(*args)


def kernel(features, edge_index, W1, b1, W2, b2, W3, b3):
    src = edge_index[0].astype(jnp.int32)
    dst = edge_index[1].astype(jnp.int32)
    srcp = src.reshape(NC, NS, NCHUNK, CHUNK)
    dstp = dst.reshape(NC, NS, NCHUNK, CHUNK)
    srcd = src.reshape(NS, ND, CD)
    dstd = dst.reshape(NS, ND, CD)
    x = jnp.pad(features, ((0, N_PAD - N_NODES), (0, 0)))

    zeros_f = jnp.zeros((N_PAD, F), jnp.float32)
    zeros_d = jnp.zeros((N_PAD, DEG_W), jnp.float32)
    ones_d = jnp.ones((CD, DEG_W), jnp.float32)
    b1r = b1[None, :]
    b2r = b2[None, :]
    W3p = jnp.pad(W3, ((0, 0), (0, F - N_CLASSES)))
    b3r = jnp.pad(b3, (0, F - N_CLASSES))[None, :]

    od, idg = _sc_degrees(srcd, dstd, ones_d, zeros_d)

    y0 = _tc(_l1_body, (N_PAD, F), x, W1, od)
    g1 = _sc_aggregate(y0, srcp, dstp, zeros_f)
    y1 = _tc(_mid_body, (N_PAD, F), g1, od, idg, b1r, W2)
    g2 = _sc_aggregate(y1, srcp, dstp, zeros_f)
    y2 = _tc(_pre3_body, (N_PAD, F), g2, od, idg, b2r)
    g3 = _sc_aggregate(y2, srcp, dstp, zeros_f)
    out = _tc(_final_body, (N_PAD, F), g3, idg, W3p, b3r)
    return out[:N_NODES, :N_CLASSES]


# speedup vs baseline: 11.1668x; 1.2001x over previous
import functools

import jax
import jax.numpy as jnp
from jax import lax
from jax.experimental import pallas as pl
from jax.experimental.pallas import tpu as pltpu
from jax.experimental.pallas import tpu_sc as plsc

N_NODES = 10000
N_PAD = 10240
N_EDGES = 320000
F = 128
N_CLASSES = 40

NC = 2
NS = 16
CHUNK = 100
NCHUNK = 100
RPS = N_PAD // NS
DEG_W = 16
CD = 200
ND = 100

_mesh = plsc.VectorSubcoreMesh(core_axis_name="c", subcore_axis_name="s")
_sc_params = pltpu.CompilerParams(use_tc_tiling_on_sc=False)


def _agg_body(y_hbm, src_hbm, dst_hbm, zeros_hbm, out_hbm,
              srcv, dstv, rows0, rows1, acc, sem0, sem1):
    c = lax.axis_index("c")
    s = lax.axis_index("s")
    pltpu.sync_copy(src_hbm.at[c].at[s], srcv)
    pltpu.sync_copy(dst_hbm.at[c].at[s], dstv)
    sl = pl.ds(s * RPS, RPS)
    pltpu.sync_copy(zeros_hbm.at[sl], acc.at[sl])
    plsc.subcore_barrier()

    pltpu.async_copy(y_hbm.at[srcv.at[0]], rows0, sem0)
    pltpu.async_copy(y_hbm.at[srcv.at[1]], rows1, sem1)

    @pl.loop(0, NCHUNK, step=2)
    def _(j):
        pltpu.make_async_copy(y_hbm.at[srcv.at[0]], rows0, sem0).wait()
        pltpu.sync_copy(rows0, acc.at[dstv.at[j]], add=True)

        @pl.when(j + 2 < NCHUNK)
        def _():
            pltpu.async_copy(y_hbm.at[srcv.at[j + 2]], rows0, sem0)

        pltpu.make_async_copy(y_hbm.at[srcv.at[0]], rows1, sem1).wait()
        pltpu.sync_copy(rows1, acc.at[dstv.at[j + 1]], add=True)

        @pl.when(j + 3 < NCHUNK)
        def _():
            pltpu.async_copy(y_hbm.at[srcv.at[j + 3]], rows1, sem1)

    plsc.subcore_barrier()
    pltpu.sync_copy(acc.at[sl], out_hbm.at[c].at[sl])


@functools.partial(
    pl.kernel, mesh=_mesh,
    out_type=jax.ShapeDtypeStruct((NC, N_PAD, F), jnp.float32),
    scratch_types=[
        pltpu.VMEM((NCHUNK, CHUNK), jnp.int32),
        pltpu.VMEM((NCHUNK, CHUNK), jnp.int32),
        pltpu.VMEM((CHUNK, F), jnp.float32),
        pltpu.VMEM((CHUNK, F), jnp.float32),
        pltpu.VMEM_SHARED((N_PAD, F), jnp.float32),
        pltpu.SemaphoreType.DMA,
        pltpu.SemaphoreType.DMA,
    ],
    compiler_params=_sc_params,
)
def _sc_aggregate(*args):
    _agg_body(*args)


def _deg_body(src_hbm, dst_hbm, ones_hbm, zeros_hbm, od_hbm, id_hbm,
              idxv, ones_v, acc):
    c = lax.axis_index("c")
    s = lax.axis_index("s")

    @pl.when(c == 0)
    def _():
        pltpu.sync_copy(src_hbm.at[s], idxv)

    @pl.when(c == 1)
    def _():
        pltpu.sync_copy(dst_hbm.at[s], idxv)
    pltpu.sync_copy(ones_hbm, ones_v)
    sl = pl.ds(s * RPS, RPS)
    pltpu.sync_copy(zeros_hbm.at[sl], acc.at[sl])
    plsc.subcore_barrier()

    @pl.loop(0, ND)
    def _(j):
        pltpu.sync_copy(ones_v, acc.at[idxv.at[j]], add=True)

    plsc.subcore_barrier()

    @pl.when(c == 0)
    def _():
        pltpu.sync_copy(acc.at[sl], od_hbm.at[sl])

    @pl.when(c == 1)
    def _():
        pltpu.sync_copy(acc.at[sl], id_hbm.at[sl])


@functools.partial(
    pl.kernel, mesh=_mesh,
    out_type=(jax.ShapeDtypeStruct((N_PAD, DEG_W), jnp.float32),
              jax.ShapeDtypeStruct((N_PAD, DEG_W), jnp.float32)),
    scratch_types=[
        pltpu.VMEM((ND, CD), jnp.int32),
        pltpu.VMEM((CD, DEG_W), jnp.float32),
        pltpu.VMEM_SHARED((N_PAD, DEG_W), jnp.float32),
    ],
    compiler_params=_sc_params,
)
def _sc_degrees(*args):
    _deg_body(*args)


def _rs(ref):
    return lax.rsqrt(jnp.maximum(ref[:, 0:1], 1.0))


def _l1_body(x_ref, w_ref, od_ref, o_ref):
    xw = jnp.dot(x_ref[...], w_ref[...], preferred_element_type=jnp.float32)
    o_ref[...] = xw * _rs(od_ref)


def _mid_body(g_ref, od_ref, id_ref, b_ref, w_ref, o_ref):
    g = g_ref[0] + g_ref[1]
    h = jnp.maximum(g * _rs(id_ref) + b_ref[...], 0.0) * _rs(od_ref)
    o_ref[...] = jnp.dot(h, w_ref[...], preferred_element_type=jnp.float32)


def _pre3_body(g_ref, od_ref, id_ref, b_ref, o_ref):
    g = g_ref[0] + g_ref[1]
    o_ref[...] = jnp.maximum(g * _rs(id_ref) + b_ref[...], 0.0) * _rs(od_ref)


def _final_body(g_ref, id_ref, w_ref, b_ref, o_ref):
    g = (g_ref[0] + g_ref[1]) * _rs(id_ref)
    o_ref[...] = (jnp.dot(g, w_ref[...], preferred_element_type=jnp.float32)
                  + b_ref[...])


def _tc(body, out_shape, *args):
    return pl.pallas_call(
        body, out_shape=jax.ShapeDtypeStruct(out_shape, jnp.float32))(*args)


def kernel(features, edge_index, W1, b1, W2, b2, W3, b3):
    src = edge_index[0].astype(jnp.int32)
    dst = edge_index[1].astype(jnp.int32)
    srcp = src.reshape(NC, NS, NCHUNK, CHUNK)
    dstp = dst.reshape(NC, NS, NCHUNK, CHUNK)
    srcd = src.reshape(NS, ND, CD)
    dstd = dst.reshape(NS, ND, CD)
    x = jnp.pad(features, ((0, N_PAD - N_NODES), (0, 0)))

    zeros_f = jnp.zeros((N_PAD, F), jnp.float32)
    zeros_d = jnp.zeros((N_PAD, DEG_W), jnp.float32)
    ones_d = jnp.ones((CD, DEG_W), jnp.float32)
    b1r = b1[None, :]
    b2r = b2[None, :]
    W3p = jnp.pad(W3, ((0, 0), (0, F - N_CLASSES)))
    b3r = jnp.pad(b3, (0, F - N_CLASSES))[None, :]

    od, idg = _sc_degrees(srcd, dstd, ones_d, zeros_d)

    y0 = _tc(_l1_body, (N_PAD, F), x, W1, od)
    g1 = _sc_aggregate(y0, srcp, dstp, zeros_f)
    y1 = _tc(_mid_body, (N_PAD, F), g1, od, idg, b1r, W2)
    g2 = _sc_aggregate(y1, srcp, dstp, zeros_f)
    y2 = _tc(_pre3_body, (N_PAD, F), g2, od, idg, b2r)
    g3 = _sc_aggregate(y2, srcp, dstp, zeros_f)
    out = _tc(_final_body, (N_PAD, F), g3, idg, W3p, b3r)
    return out[:N_NODES, :N_CLASSES]


# speedup vs baseline: 12.4901x; 1.1185x over previous
import functools

import jax
import jax.numpy as jnp
from jax import lax
from jax.experimental import pallas as pl
from jax.experimental.pallas import tpu as pltpu
from jax.experimental.pallas import tpu_sc as plsc

N_NODES = 10000
N_PAD = 10240
N_EDGES = 320000
F = 128
N_CLASSES = 40

NC = 2
NS = 16
CHUNK = 50
NCHUNK = 200
NSTREAM = 4
RPS = N_PAD // NS
DEG_W = 16
CD = 200
ND = 100

_mesh = plsc.VectorSubcoreMesh(core_axis_name="c", subcore_axis_name="s")
_sc_params = pltpu.CompilerParams(use_tc_tiling_on_sc=False)


def _agg_body(y_hbm, src_hbm, dst_hbm, zeros_hbm, out_hbm,
              srcv, dstv, rows0, rows1, rows2, rows3, acc,
              sem0, sem1, sem2, sem3):
    c = lax.axis_index("c")
    s = lax.axis_index("s")
    pltpu.sync_copy(src_hbm.at[c].at[s], srcv)
    pltpu.sync_copy(dst_hbm.at[c].at[s], dstv)
    sl = pl.ds(s * RPS, RPS)
    pltpu.sync_copy(zeros_hbm.at[sl], acc.at[sl])
    plsc.subcore_barrier()

    streams = ((rows0, sem0), (rows1, sem1), (rows2, sem2), (rows3, sem3))
    for k, (rows, sem) in enumerate(streams):
        pltpu.async_copy(y_hbm.at[srcv.at[k]], rows, sem)

    @pl.loop(0, NCHUNK, step=NSTREAM)
    def _(j):
        for k, (rows, sem) in enumerate(streams):
            pltpu.make_async_copy(y_hbm.at[srcv.at[0]], rows, sem).wait()
            pltpu.sync_copy(rows, acc.at[dstv.at[j + k]], add=True)

            @pl.when(j + k + NSTREAM < NCHUNK)
            def _(rows=rows, sem=sem, k=k):
                pltpu.async_copy(
                    y_hbm.at[srcv.at[j + k + NSTREAM]], rows, sem)

    plsc.subcore_barrier()
    pltpu.sync_copy(acc.at[sl], out_hbm.at[c].at[sl])


@functools.partial(
    pl.kernel, mesh=_mesh,
    out_type=jax.ShapeDtypeStruct((NC, N_PAD, F), jnp.float32),
    scratch_types=[
        pltpu.VMEM((NCHUNK, CHUNK), jnp.int32),
        pltpu.VMEM((NCHUNK, CHUNK), jnp.int32),
        pltpu.VMEM((CHUNK, F), jnp.float32),
        pltpu.VMEM((CHUNK, F), jnp.float32),
        pltpu.VMEM((CHUNK, F), jnp.float32),
        pltpu.VMEM((CHUNK, F), jnp.float32),
        pltpu.VMEM_SHARED((N_PAD, F), jnp.float32),
        pltpu.SemaphoreType.DMA,
        pltpu.SemaphoreType.DMA,
        pltpu.SemaphoreType.DMA,
        pltpu.SemaphoreType.DMA,
    ],
    compiler_params=_sc_params,
)
def _sc_aggregate(*args):
    _agg_body(*args)


def _deg_body(src_hbm, dst_hbm, ones_hbm, zeros_hbm, od_hbm, id_hbm,
              idxv, ones_v, acc):
    c = lax.axis_index("c")
    s = lax.axis_index("s")

    @pl.when(c == 0)
    def _():
        pltpu.sync_copy(src_hbm.at[s], idxv)

    @pl.when(c == 1)
    def _():
        pltpu.sync_copy(dst_hbm.at[s], idxv)
    pltpu.sync_copy(ones_hbm, ones_v)
    sl = pl.ds(s * RPS, RPS)
    pltpu.sync_copy(zeros_hbm.at[sl], acc.at[sl])
    plsc.subcore_barrier()

    @pl.loop(0, ND)
    def _(j):
        pltpu.sync_copy(ones_v, acc.at[idxv.at[j]], add=True)

    plsc.subcore_barrier()

    @pl.when(c == 0)
    def _():
        pltpu.sync_copy(acc.at[sl], od_hbm.at[sl])

    @pl.when(c == 1)
    def _():
        pltpu.sync_copy(acc.at[sl], id_hbm.at[sl])


@functools.partial(
    pl.kernel, mesh=_mesh,
    out_type=(jax.ShapeDtypeStruct((N_PAD, DEG_W), jnp.float32),
              jax.ShapeDtypeStruct((N_PAD, DEG_W), jnp.float32)),
    scratch_types=[
        pltpu.VMEM((ND, CD), jnp.int32),
        pltpu.VMEM((CD, DEG_W), jnp.float32),
        pltpu.VMEM_SHARED((N_PAD, DEG_W), jnp.float32),
    ],
    compiler_params=_sc_params,
)
def _sc_degrees(*args):
    _deg_body(*args)


def _rs(ref):
    return lax.rsqrt(jnp.maximum(ref[:, 0:1], 1.0))


def _l1_body(x_ref, w_ref, od_ref, o_ref):
    xw = jnp.dot(x_ref[...], w_ref[...], preferred_element_type=jnp.float32)
    o_ref[...] = xw * _rs(od_ref)


def _mid_body(g_ref, od_ref, id_ref, b_ref, w_ref, o_ref):
    g = g_ref[0] + g_ref[1]
    h = jnp.maximum(g * _rs(id_ref) + b_ref[...], 0.0) * _rs(od_ref)
    o_ref[...] = jnp.dot(h, w_ref[...], preferred_element_type=jnp.float32)


def _pre3_body(g_ref, od_ref, id_ref, b_ref, o_ref):
    g = g_ref[0] + g_ref[1]
    o_ref[...] = jnp.maximum(g * _rs(id_ref) + b_ref[...], 0.0) * _rs(od_ref)


def _final_body(g_ref, id_ref, w_ref, b_ref, o_ref):
    g = (g_ref[0] + g_ref[1]) * _rs(id_ref)
    o_ref[...] = (jnp.dot(g, w_ref[...], preferred_element_type=jnp.float32)
                  + b_ref[...])


def _tc(body, out_shape, *args):
    return pl.pallas_call(
        body, out_shape=jax.ShapeDtypeStruct(out_shape, jnp.float32))(*args)


def kernel(features, edge_index, W1, b1, W2, b2, W3, b3):
    src = edge_index[0].astype(jnp.int32)
    dst = edge_index[1].astype(jnp.int32)
    srcp = src.reshape(NC, NS, NCHUNK, CHUNK)
    dstp = dst.reshape(NC, NS, NCHUNK, CHUNK)
    srcd = src.reshape(NS, ND, CD)
    dstd = dst.reshape(NS, ND, CD)
    x = jnp.pad(features, ((0, N_PAD - N_NODES), (0, 0)))

    zeros_f = jnp.zeros((N_PAD, F), jnp.float32)
    zeros_d = jnp.zeros((N_PAD, DEG_W), jnp.float32)
    ones_d = jnp.ones((CD, DEG_W), jnp.float32)
    b1r = b1[None, :]
    b2r = b2[None, :]
    W3p = jnp.pad(W3, ((0, 0), (0, F - N_CLASSES)))
    b3r = jnp.pad(b3, (0, F - N_CLASSES))[None, :]

    od, idg = _sc_degrees(srcd, dstd, ones_d, zeros_d)

    y0 = _tc(_l1_body, (N_PAD, F), x, W1, od)
    g1 = _sc_aggregate(y0, srcp, dstp, zeros_f)
    y1 = _tc(_mid_body, (N_PAD, F), g1, od, idg, b1r, W2)
    g2 = _sc_aggregate(y1, srcp, dstp, zeros_f)
    y2 = _tc(_pre3_body, (N_PAD, F), g2, od, idg, b2r)
    g3 = _sc_aggregate(y2, srcp, dstp, zeros_f)
    out = _tc(_final_body, (N_PAD, F), g3, idg, W3p, b3r)
    return out[:N_NODES, :N_CLASSES]


# speedup vs baseline: 12.7024x; 1.0170x over previous
import functools

import jax
import jax.numpy as jnp
from jax import lax
from jax.experimental import pallas as pl
from jax.experimental.pallas import tpu as pltpu
from jax.experimental.pallas import tpu_sc as plsc

N_NODES = 10000
N_PAD = 10240
N_EDGES = 320000
F = 128
N_CLASSES = 40

NC = 2
NS = 16
CHUNK = 40
NCHUNK = 250
NSTREAM = 5
RPS = N_PAD // NS
DEG_W = 16
CD = 200
ND = 100

_mesh = plsc.VectorSubcoreMesh(core_axis_name="c", subcore_axis_name="s")
_sc_params = pltpu.CompilerParams(use_tc_tiling_on_sc=False)


def _agg_body(y_hbm, src_hbm, dst_hbm, zeros_hbm, out_hbm,
              srcv, dstv, rows0, rows1, rows2, rows3, rows4, acc,
              sem0, sem1, sem2, sem3, sem4):
    c = lax.axis_index("c")
    s = lax.axis_index("s")
    pltpu.sync_copy(src_hbm.at[c].at[s], srcv)
    pltpu.sync_copy(dst_hbm.at[c].at[s], dstv)
    sl = pl.ds(s * RPS, RPS)
    pltpu.sync_copy(zeros_hbm.at[sl], acc.at[sl])
    plsc.subcore_barrier()

    streams = ((rows0, sem0), (rows1, sem1), (rows2, sem2), (rows3, sem3),
               (rows4, sem4))
    for k, (rows, sem) in enumerate(streams):
        pltpu.async_copy(y_hbm.at[srcv.at[k]], rows, sem)

    @pl.loop(0, NCHUNK, step=NSTREAM)
    def _(j):
        for k, (rows, sem) in enumerate(streams):
            pltpu.make_async_copy(y_hbm.at[srcv.at[0]], rows, sem).wait()
            pltpu.sync_copy(rows, acc.at[dstv.at[j + k]], add=True)

            @pl.when(j + k + NSTREAM < NCHUNK)
            def _(rows=rows, sem=sem, k=k):
                pltpu.async_copy(
                    y_hbm.at[srcv.at[j + k + NSTREAM]], rows, sem)

    plsc.subcore_barrier()
    pltpu.sync_copy(acc.at[sl], out_hbm.at[c].at[sl])


@functools.partial(
    pl.kernel, mesh=_mesh,
    out_type=jax.ShapeDtypeStruct((NC, N_PAD, F), jnp.float32),
    scratch_types=[
        pltpu.VMEM((NCHUNK, CHUNK), jnp.int32),
        pltpu.VMEM((NCHUNK, CHUNK), jnp.int32),
        pltpu.VMEM((CHUNK, F), jnp.float32),
        pltpu.VMEM((CHUNK, F), jnp.float32),
        pltpu.VMEM((CHUNK, F), jnp.float32),
        pltpu.VMEM((CHUNK, F), jnp.float32),
        pltpu.VMEM((CHUNK, F), jnp.float32),
        pltpu.VMEM_SHARED((N_PAD, F), jnp.float32),
        pltpu.SemaphoreType.DMA,
        pltpu.SemaphoreType.DMA,
        pltpu.SemaphoreType.DMA,
        pltpu.SemaphoreType.DMA,
        pltpu.SemaphoreType.DMA,
    ],
    compiler_params=_sc_params,
)
def _sc_aggregate(*args):
    _agg_body(*args)


def _deg_body(src_hbm, dst_hbm, ones_hbm, zeros_hbm, od_hbm, id_hbm,
              idxv, ones_v, acc):
    c = lax.axis_index("c")
    s = lax.axis_index("s")

    @pl.when(c == 0)
    def _():
        pltpu.sync_copy(src_hbm.at[s], idxv)

    @pl.when(c == 1)
    def _():
        pltpu.sync_copy(dst_hbm.at[s], idxv)
    pltpu.sync_copy(ones_hbm, ones_v)
    sl = pl.ds(s * RPS, RPS)
    pltpu.sync_copy(zeros_hbm.at[sl], acc.at[sl])
    plsc.subcore_barrier()

    @pl.loop(0, ND)
    def _(j):
        pltpu.sync_copy(ones_v, acc.at[idxv.at[j]], add=True)

    plsc.subcore_barrier()

    @pl.when(c == 0)
    def _():
        pltpu.sync_copy(acc.at[sl], od_hbm.at[sl])

    @pl.when(c == 1)
    def _():
        pltpu.sync_copy(acc.at[sl], id_hbm.at[sl])


@functools.partial(
    pl.kernel, mesh=_mesh,
    out_type=(jax.ShapeDtypeStruct((N_PAD, DEG_W), jnp.float32),
              jax.ShapeDtypeStruct((N_PAD, DEG_W), jnp.float32)),
    scratch_types=[
        pltpu.VMEM((ND, CD), jnp.int32),
        pltpu.VMEM((CD, DEG_W), jnp.float32),
        pltpu.VMEM_SHARED((N_PAD, DEG_W), jnp.float32),
    ],
    compiler_params=_sc_params,
)
def _sc_degrees(*args):
    _deg_body(*args)


def _rs(ref):
    return lax.rsqrt(jnp.maximum(ref[:, 0:1], 1.0))


def _l1_body(x_ref, w_ref, od_ref, o_ref):
    xw = jnp.dot(x_ref[...], w_ref[...], preferred_element_type=jnp.float32)
    o_ref[...] = xw * _rs(od_ref)


def _mid_body(g_ref, od_ref, id_ref, b_ref, w_ref, o_ref):
    g = g_ref[0] + g_ref[1]
    h = jnp.maximum(g * _rs(id_ref) + b_ref[...], 0.0) * _rs(od_ref)
    o_ref[...] = jnp.dot(h, w_ref[...], preferred_element_type=jnp.float32)


def _pre3_body(g_ref, od_ref, id_ref, b_ref, o_ref):
    g = g_ref[0] + g_ref[1]
    o_ref[...] = jnp.maximum(g * _rs(id_ref) + b_ref[...], 0.0) * _rs(od_ref)


def _final_body(g_ref, id_ref, w_ref, b_ref, o_ref):
    g = (g_ref[0] + g_ref[1]) * _rs(id_ref)
    o_ref[...] = (jnp.dot(g, w_ref[...], preferred_element_type=jnp.float32)
                  + b_ref[...])


def _tc(body, out_shape, *args):
    return pl.pallas_call(
        body, out_shape=jax.ShapeDtypeStruct(out_shape, jnp.float32))(*args)


def kernel(features, edge_index, W1, b1, W2, b2, W3, b3):
    src = edge_index[0].astype(jnp.int32)
    dst = edge_index[1].astype(jnp.int32)
    srcp = src.reshape(NC, NS, NCHUNK, CHUNK)
    dstp = dst.reshape(NC, NS, NCHUNK, CHUNK)
    srcd = src.reshape(NS, ND, CD)
    dstd = dst.reshape(NS, ND, CD)
    x = jnp.pad(features, ((0, N_PAD - N_NODES), (0, 0)))

    zeros_f = jnp.zeros((N_PAD, F), jnp.float32)
    zeros_d = jnp.zeros((N_PAD, DEG_W), jnp.float32)
    ones_d = jnp.ones((CD, DEG_W), jnp.float32)
    b1r = b1[None, :]
    b2r = b2[None, :]
    W3p = jnp.pad(W3, ((0, 0), (0, F - N_CLASSES)))
    b3r = jnp.pad(b3, (0, F - N_CLASSES))[None, :]

    od, idg = _sc_degrees(srcd, dstd, ones_d, zeros_d)

    y0 = _tc(_l1_body, (N_PAD, F), x, W1, od)
    g1 = _sc_aggregate(y0, srcp, dstp, zeros_f)
    y1 = _tc(_mid_body, (N_PAD, F), g1, od, idg, b1r, W2)
    g2 = _sc_aggregate(y1, srcp, dstp, zeros_f)
    y2 = _tc(_pre3_body, (N_PAD, F), g2, od, idg, b2r)
    g3 = _sc_aggregate(y2, srcp, dstp, zeros_f)
    out = _tc(_final_body, (N_PAD, F), g3, idg, W3p, b3r)
    return out[:N_NODES, :N_CLASSES]
